# baseline (speedup 1.0000x reference)
import functools

import jax
import jax.numpy as jnp
from jax.experimental import pallas as pl
from jax.experimental.pallas import tpu as pltpu


def _fused4t_kernel(n_lora, *refs):
    x_ref = refs[0]
    ws = refs[1:5]
    ats = refs[5:5 + n_lora]
    bts = refs[5 + n_lora:5 + 2 * n_lora]
    y_ref = refs[-1]

    half = x_ref.shape[1] // 2
    ts = [x_ref[:, :half], x_ref[:, half:]]
    for m in range(4):
        w = ws[m][...]
        for c in range(2):
            t = ts[c]
            acc = jnp.dot(w, t, preferred_element_type=jnp.float32)
            if m < n_lora:
                u = jnp.dot(ats[m][...], t, preferred_element_type=jnp.float32)
                acc += jnp.dot(bts[m][...].astype(jnp.float32), u,
                               preferred_element_type=jnp.float32)
            ts[c] = (t.astype(jnp.float32) + acc).astype(t.dtype)
    y_ref[:, :half] = ts[0]
    y_ref[:, half:] = ts[1]


def _fused4t(t, ws, lora_ats, lora_bts, *, tn=512):
    H, M = t.shape
    n_lora = len(lora_ats)
    tn = min(tn, M)
    grid = (M // tn,)

    full = lambda shape: pl.BlockSpec(shape, lambda i: (0,) * len(shape))
    in_specs = [pl.BlockSpec((H, tn), lambda i: (0, i))]
    in_specs += [full((H, H))] * 4
    in_specs += [full(a.shape) for a in lora_ats]
    in_specs += [full(b.shape) for b in lora_bts]

    R = lora_ats[0].shape[0] if lora_ats else 0
    cost = pl.CostEstimate(
        flops=4 * 2 * M * H * H + n_lora * (2 * M * H * R + 2 * M * R * H),
        transcendentals=0,
        bytes_accessed=2 * (2 * M * H + 4 * H * H))

    return pl.pallas_call(
        functools.partial(_fused4t_kernel, n_lora),
        out_shape=jax.ShapeDtypeStruct((H, M), t.dtype),
        grid=grid,
        in_specs=in_specs,
        out_specs=pl.BlockSpec((H, tn), lambda i: (0, i)),
        compiler_params=pltpu.CompilerParams(
            dimension_semantics=("parallel",),
            vmem_limit_bytes=100 * 1024 * 1024),
        cost_estimate=cost,
    )(t, *ws, *lora_ats, *lora_bts)


def kernel(input_ids, embed, layers_0_q_proj_weight, layers_0_q_proj_lora_A, layers_0_q_proj_lora_B, layers_0_v_proj_weight, layers_0_v_proj_lora_A, layers_0_v_proj_lora_B, layers_1_q_proj_weight, layers_1_q_proj_lora_A, layers_1_q_proj_lora_B, layers_1_v_proj_weight, layers_1_v_proj_lora_A, layers_1_v_proj_lora_B, layers_2_q_proj_weight, layers_2_q_proj_lora_A, layers_2_q_proj_lora_B, layers_2_v_proj_weight, layers_2_v_proj_lora_A, layers_2_v_proj_lora_B, layers_3_q_proj_weight, layers_3_q_proj_lora_A, layers_3_q_proj_lora_B, layers_3_v_proj_weight, layers_3_v_proj_lora_A, layers_3_v_proj_lora_B):
    B, S = input_ids.shape
    H = embed.shape[1]
    t = embed[input_ids].reshape(B * S, H).T

    t = _fused4t(
        t,
        [layers_0_q_proj_weight, layers_0_v_proj_weight,
         layers_1_q_proj_weight, layers_1_v_proj_weight],
        [layers_0_q_proj_lora_A.T, layers_0_v_proj_lora_A.T,
         layers_1_q_proj_lora_A.T, layers_1_v_proj_lora_A.T],
        [layers_0_q_proj_lora_B.T, layers_0_v_proj_lora_B.T,
         layers_1_q_proj_lora_B.T, layers_1_v_proj_lora_B.T])

    t = _fused4t(
        t,
        [layers_2_q_proj_weight, layers_2_v_proj_weight,
         layers_3_q_proj_weight, layers_3_v_proj_weight],
        [layers_2_q_proj_lora_A.T, layers_2_v_proj_lora_A.T],
        [layers_2_q_proj_lora_B.T, layers_2_v_proj_lora_B.T])

    return t.T.reshape(B, S, H)


# speedup vs baseline: 1.0313x; 1.0313x over previous
import functools

import jax
import jax.numpy as jnp
from jax.experimental import pallas as pl
from jax.experimental.pallas import tpu as pltpu


def _eye(n, dtype):
    r = jax.lax.broadcasted_iota(jnp.int32, (n, n), 0)
    c = jax.lax.broadcasted_iota(jnp.int32, (n, n), 1)
    return (r == c).astype(dtype)


def _fused4t_kernel(n_lora, first, last, *refs):
    x_ref = refs[0]
    ws = refs[1:5]
    ats = refs[5:5 + n_lora]
    bts = refs[5 + n_lora:5 + 2 * n_lora]
    y_ref = refs[-1]

    if first:
        tm = x_ref.shape[0]
        ident = _eye(tm, x_ref.dtype)
        t_in = jax.lax.dot_general(
            x_ref[...], ident, (((0,), (0,)), ((), ())),
            preferred_element_type=jnp.float32).astype(x_ref.dtype)
    else:
        tm = x_ref.shape[1]
        t_in = x_ref[...]

    half = tm // 2
    ts = [t_in[:, :half], t_in[:, half:]]
    for m in range(4):
        w = ws[m][...]
        for c in range(2):
            t = ts[c]
            acc = jnp.dot(w, t, preferred_element_type=jnp.float32)
            if m < n_lora:
                u = jnp.dot(ats[m][...], t, preferred_element_type=jnp.float32)
                acc += jnp.dot(bts[m][...].astype(jnp.float32), u,
                               preferred_element_type=jnp.float32)
            ts[c] = (t.astype(jnp.float32) + acc).astype(t.dtype)

    if last:
        ident = _eye(half, y_ref.dtype)
        for c, sl in ((0, slice(0, half)), (1, slice(half, tm))):
            y_ref[sl, :] = jax.lax.dot_general(
                ident, ts[c], (((1,), (1,)), ((), ())),
                preferred_element_type=jnp.float32).astype(y_ref.dtype)
    else:
        y_ref[:, :half] = ts[0]
        y_ref[:, half:] = ts[1]


def _fused4t(x, ws, lora_ats, lora_bts, *, tm=512, first=False, last=False):
    if first:
        M, H = x.shape
    else:
        H, M = x.shape
    n_lora = len(lora_ats)
    tm = min(tm, M)
    grid = (M // tm,)

    full = lambda shape: pl.BlockSpec(shape, lambda i: (0,) * len(shape))
    if first:
        in_specs = [pl.BlockSpec((tm, H), lambda i: (i, 0))]
    else:
        in_specs = [pl.BlockSpec((H, tm), lambda i: (0, i))]
    in_specs += [full((H, H))] * 4
    in_specs += [full(a.shape) for a in lora_ats]
    in_specs += [full(b.shape) for b in lora_bts]

    if last:
        out_shape = jax.ShapeDtypeStruct((M, H), x.dtype)
        out_specs = pl.BlockSpec((tm, H), lambda i: (i, 0))
    else:
        out_shape = jax.ShapeDtypeStruct((H, M), x.dtype)
        out_specs = pl.BlockSpec((H, tm), lambda i: (0, i))

    R = lora_ats[0].shape[0] if lora_ats else 0
    cost = pl.CostEstimate(
        flops=4 * 2 * M * H * H + n_lora * (2 * M * H * R + 2 * M * R * H),
        transcendentals=0,
        bytes_accessed=2 * (2 * M * H + 4 * H * H))

    return pl.pallas_call(
        functools.partial(_fused4t_kernel, n_lora, first, last),
        out_shape=out_shape,
        grid=grid,
        in_specs=in_specs,
        out_specs=out_specs,
        compiler_params=pltpu.CompilerParams(
            dimension_semantics=("parallel",),
            vmem_limit_bytes=100 * 1024 * 1024),
        cost_estimate=cost,
    )(x, *ws, *lora_ats, *lora_bts)


def kernel(input_ids, embed, layers_0_q_proj_weight, layers_0_q_proj_lora_A, layers_0_q_proj_lora_B, layers_0_v_proj_weight, layers_0_v_proj_lora_A, layers_0_v_proj_lora_B, layers_1_q_proj_weight, layers_1_q_proj_lora_A, layers_1_q_proj_lora_B, layers_1_v_proj_weight, layers_1_v_proj_lora_A, layers_1_v_proj_lora_B, layers_2_q_proj_weight, layers_2_q_proj_lora_A, layers_2_q_proj_lora_B, layers_2_v_proj_weight, layers_2_v_proj_lora_A, layers_2_v_proj_lora_B, layers_3_q_proj_weight, layers_3_q_proj_lora_A, layers_3_q_proj_lora_B, layers_3_v_proj_weight, layers_3_v_proj_lora_A, layers_3_v_proj_lora_B):
    B, S = input_ids.shape
    H = embed.shape[1]
    h = embed[input_ids].reshape(B * S, H)

    t = _fused4t(
        h,
        [layers_0_q_proj_weight, layers_0_v_proj_weight,
         layers_1_q_proj_weight, layers_1_v_proj_weight],
        [layers_0_q_proj_lora_A.T, layers_0_v_proj_lora_A.T,
         layers_1_q_proj_lora_A.T, layers_1_v_proj_lora_A.T],
        [layers_0_q_proj_lora_B.T, layers_0_v_proj_lora_B.T,
         layers_1_q_proj_lora_B.T, layers_1_v_proj_lora_B.T],
        first=True)

    y = _fused4t(
        t,
        [layers_2_q_proj_weight, layers_2_v_proj_weight,
         layers_3_q_proj_weight, layers_3_v_proj_weight],
        [layers_2_q_proj_lora_A.T, layers_2_v_proj_lora_A.T],
        [layers_2_q_proj_lora_B.T, layers_2_v_proj_lora_B.T],
        last=True)

    return y.reshape(B, S, H)
